# baseline (speedup 1.0000x reference)
import functools

import jax
import jax.numpy as jnp
from jax import lax
from jax.experimental import pallas as pl
from jax.experimental.pallas import tpu as pltpu
from jax.experimental.pallas import tpu_sc as plsc

_NC = 2
_NS = 16
_NW = _NC * _NS
_LANE = 128
_SUB = 8


def _emb_grid(batch, seq, d_model):
    n_bg = batch // _LANE
    n_dg = d_model // _SUB
    mesh = plsc.VectorSubcoreMesh(core_axis_name="c", subcore_axis_name="s")

    @functools.partial(
        pl.kernel,
        mesh=mesh,
        out_type=jax.ShapeDtypeStruct((seq, n_dg, n_bg, _SUB, _LANE), jnp.float32),
        scratch_types=[
            pltpu.VMEM((2, _LANE), jnp.int32),
            pltpu.VMEM((2, _LANE, d_model), jnp.float32),
            pltpu.VMEM((2, n_dg, _SUB, _LANE), jnp.float32),
            pltpu.SemaphoreType.DMA,
            pltpu.SemaphoreType.DMA,
            pltpu.SemaphoreType.DMA,
            pltpu.SemaphoreType.DMA,
        ],
        compiler_params=pltpu.CompilerParams(
            use_tc_tiling_on_sc=False, needs_layout_passes=False
        ),
    )
    def emb(idx_hbm, table_hbm, out_hbm, tok_v, rows_v, tiles_v,
            g0, g1, s0, s1):
        wid = lax.axis_index("s") * _NC + lax.axis_index("c")
        gsem = (g0, g1)
        ssem = (s0, s1)

        def fire_gather(s, slot):
            pltpu.sync_copy(
                idx_hbm.at[pl.ds(s * batch + wid * _LANE, _LANE)],
                tok_v.at[slot],
            )
            pltpu.async_copy(
                table_hbm.at[tok_v.at[slot]], rows_v.at[slot], gsem[slot]
            )

        def wait_gather(slot):
            pltpu.make_async_copy(
                table_hbm.at[pl.ds(0, _LANE)], rows_v.at[slot], gsem[slot]
            ).wait()

        def transpose(slot):
            def jstep(j, carry):
                rowi = lax.iota(jnp.int32, 16) + j * 16
                for dg in range(n_dg):
                    for sub in range(_SUB):
                        col = jnp.full((16,), dg * _SUB + sub, jnp.int32)
                        v = plsc.load_gather(rows_v.at[slot], [rowi, col])
                        tiles_v[slot, dg, sub, pl.ds(j * 16, 16)] = v
                return carry

            lax.fori_loop(0, _LANE // 16, jstep, 0)

        def fire_store(s, slot):
            pltpu.async_copy(
                tiles_v.at[slot], out_hbm.at[s, :, wid], ssem[slot]
            )

        def wait_store(slot):
            pltpu.make_async_copy(
                tiles_v.at[slot], out_hbm.at[0, :, wid], ssem[slot]
            ).wait()

        fire_gather(0, 0)
        fire_gather(1, 1)
        for k in range(2):
            wait_gather(k)
            transpose(k)
            fire_gather(k + 2, k)
            fire_store(k, k)

        def superstep(t, carry):
            for b in range(2):
                k = 2 * t + b
                wait_store(b)
                wait_gather(b)
                transpose(b)
                @pl.when(k + 2 < seq)
                def _():
                    fire_gather(k + 2, b)
                fire_store(k, b)
            return carry

        lax.fori_loop(1, seq // 2, superstep, 0)
        wait_store(0)
        wait_store(1)

    return emb


def kernel(tokens, token_emb):
    b, s = tokens.shape
    v, d = token_emb.shape
    idx1d = jnp.reshape(tokens.T, (b * s,)).astype(jnp.int32)
    out5 = _emb_grid(b, s, d)(idx1d, token_emb)
    return jnp.transpose(out5, (2, 4, 0, 1, 3)).reshape(b, s, d)


# speedup vs baseline: 1.5112x; 1.5112x over previous
import functools

import jax
import jax.numpy as jnp
from jax import lax
from jax.experimental import pallas as pl
from jax.experimental.pallas import tpu as pltpu
from jax.experimental.pallas import tpu_sc as plsc

_NC = 2
_NS = 16
_NW = _NC * _NS
_LANE = 128
_SUB = 8


def _emb_grid(batch, seq, d_model):
    n_bg = batch // _LANE
    n_dg = d_model // _SUB
    mesh = plsc.VectorSubcoreMesh(core_axis_name="c", subcore_axis_name="s")

    @functools.partial(
        pl.kernel,
        mesh=mesh,
        out_type=jax.ShapeDtypeStruct((seq, n_dg, n_bg, _SUB, _LANE), jnp.float32),
        scratch_types=[
            pltpu.VMEM((seq, _LANE), jnp.int32),
            pltpu.VMEM((2, _LANE, d_model), jnp.float32),
            pltpu.VMEM((2, n_dg, _SUB, _LANE), jnp.float32),
            pltpu.SemaphoreType.DMA,
            pltpu.SemaphoreType.DMA,
            pltpu.SemaphoreType.DMA,
            pltpu.SemaphoreType.DMA,
        ],
        compiler_params=pltpu.CompilerParams(
            use_tc_tiling_on_sc=False, needs_layout_passes=False
        ),
    )
    def emb(idx_hbm, table_hbm, out_hbm, idx_v, rows_v, tiles_v,
            g0, g1, s0, s1):
        wid = lax.axis_index("s") * _NC + lax.axis_index("c")
        gsem = (g0, g1)
        ssem = (s0, s1)

        pltpu.sync_copy(idx_hbm.at[wid], idx_v)

        def fire_gather(s, slot):
            pltpu.async_copy(
                table_hbm.at[idx_v.at[s]], rows_v.at[slot], gsem[slot]
            )

        def wait_gather(slot):
            pltpu.make_async_copy(
                table_hbm.at[pl.ds(0, _LANE)], rows_v.at[slot], gsem[slot]
            ).wait()

        def transpose(slot):
            @plsc.parallel_loop(0, (_LANE // 16) * d_model, unroll=8)
            def _(i):
                j = lax.shift_right_logical(i, 6)
                c = lax.bitwise_and(i, d_model - 1)
                dg = lax.shift_right_logical(c, 3)
                sub = lax.bitwise_and(c, _SUB - 1)
                rowi = lax.iota(jnp.int32, 16) + j * 16
                colv = jnp.zeros((16,), jnp.int32) + c
                v = plsc.load_gather(rows_v.at[slot], [rowi, colv])
                tiles_v[slot, dg, sub, pl.ds(j * 16, 16)] = v

        def fire_store(s, slot):
            pltpu.async_copy(
                tiles_v.at[slot], out_hbm.at[s, :, wid], ssem[slot]
            )

        def wait_store(slot):
            pltpu.make_async_copy(
                tiles_v.at[slot], out_hbm.at[0, :, wid], ssem[slot]
            ).wait()

        fire_gather(0, 0)
        fire_gather(1, 1)
        for k in range(2):
            wait_gather(k)
            transpose(k)
            fire_gather(k + 2, k)
            fire_store(k, k)

        def superstep(t, carry):
            for b in range(2):
                k = 2 * t + b
                wait_store(b)
                wait_gather(b)
                transpose(b)
                @pl.when(k + 2 < seq)
                def _():
                    fire_gather(k + 2, b)
                fire_store(k, b)
            return carry

        lax.fori_loop(1, seq // 2, superstep, 0)
        wait_store(0)
        wait_store(1)

    return emb


def kernel(tokens, token_emb):
    b, s = tokens.shape
    v, d = token_emb.shape
    idx3 = jnp.transpose(
        tokens.reshape(_NW, _LANE, s), (0, 2, 1)
    ).astype(jnp.int32)
    out5 = _emb_grid(b, s, d)(idx3, token_emb)
    return jnp.transpose(out5, (2, 4, 0, 1, 3)).reshape(b, s, d)
